# baseline (speedup 1.0000x reference)
import functools

import jax
import jax.numpy as jnp
from jax import lax
from jax.experimental import pallas as pl
from jax.experimental.pallas import tpu as pltpu
from jax.experimental.pallas import tpu_sc as plsc

_N = 10000
_E = 160000
_H = 8
_D = 64
_HID = _H * _D
_IN = 256

_NP = 10240
_BM = 1024

_NTILE = 32
_BLK = 160
_TRASH = _BLK
_ECH = 2000


def _qkv_body(h_ref, w_ref, b_ref, q_ref, k_ref, v_ref):
    r = jnp.dot(h_ref[...], w_ref[...], preferred_element_type=jnp.float32)
    r = r + b_ref[...]
    q_ref[...] = r[:, :_HID].astype(jnp.bfloat16)
    k_ref[...] = r[:, _HID:2 * _HID].astype(jnp.bfloat16)
    v_ref[...] = r[:, 2 * _HID:]


_mesh = plsc.VectorSubcoreMesh(core_axis_name="c", subcore_axis_name="s")


@functools.partial(
    pl.kernel,
    out_type=jax.ShapeDtypeStruct((_N, _HID), jnp.float32),
    mesh=_mesh,
    compiler_params=pltpu.CompilerParams(needs_layout_passes=False),
    scratch_types=[
        pltpu.VMEM((_ECH,), jnp.int32),
        pltpu.VMEM((_ECH,), jnp.int32),
        pltpu.VMEM((_ECH + 32,), jnp.int32),
        pltpu.VMEM((_ECH + 32,), jnp.int32),
        pltpu.VMEM((_ECH + 32,), jnp.int32),
        pltpu.VMEM((2, 16), jnp.int32),
        pltpu.VMEM((2, 16), jnp.int32),
        pltpu.VMEM((2, 16, _HID // 2), jnp.int32),
        pltpu.VMEM((2, 16, _HID // 2), jnp.int32),
        pltpu.VMEM((2, 16, _HID), jnp.float32),
        pltpu.VMEM(((_BLK + 1) * _HID,), jnp.float32),
        pltpu.VMEM(((_BLK + 1) * 16,), jnp.float32),
        pltpu.SemaphoreType.DMA,
        pltpu.SemaphoreType.DMA,
    ],
)
def _edge_kernel(q_hbm, k_hbm, v_hbm, src_hbm, dst_hbm, out_hbm,
                 src_buf, dst_buf, lsrc, ldst, lslot,
                 idx_src_d, idx_dst_d,
                 krows_d, qrows_d, vrows_d,
                 wv_acc, z_acc, sem0, sem1):
    cid = lax.axis_index("c")
    sid = lax.axis_index("s")
    wid = cid * 16 + sid
    lane = jnp.arange(16, dtype=jnp.int32)
    z16f = jnp.zeros((16,), jnp.float32)
    dump = jnp.full((16,), _ECH + 16, jnp.int32)
    rots = [(lane + r) & 15 for r in (8, 4, 2, 1)]
    hvecs = [jnp.full((16,), 1, jnp.int32) * h for h in range(_H)]

    def _allsum(v):
        for r in rots:
            v = v + v.at[r].get(mode='promise_in_bounds')
        return v

    def _issue(P, bi):
        base = bi * 16
        idx_src_d[P, :] = lsrc[pl.ds(base, 16)]
        idx_dst_d[P, :] = ldst[pl.ds(base, 16)]
        s = sem0 if P == 0 else sem1
        pltpu.async_copy(k_hbm.at[idx_src_d.at[P]], krows_d.at[P], s)
        pltpu.async_copy(q_hbm.at[idx_dst_d.at[P]], qrows_d.at[P], s)
        pltpu.async_copy(v_hbm.at[idx_src_d.at[P]], vrows_d.at[P], s)

    def _wait(P):
        s = sem0 if P == 0 else sem1
        pltpu.make_async_copy(k_hbm.at[idx_src_d.at[P]], krows_d.at[P], s).wait()
        pltpu.make_async_copy(q_hbm.at[idx_dst_d.at[P]], qrows_d.at[P], s).wait()
        pltpu.make_async_copy(v_hbm.at[idx_src_d.at[P]], vrows_d.at[P], s).wait()

    for bp in range(2):
        wbase = (bp * _NTILE + wid) * _BLK
        hi_all = jnp.minimum(wbase + _BLK, _N)
        lo_v = jnp.full((16,), 1, jnp.int32) * wbase
        hi_v = jnp.full((16,), 1, jnp.int32) * hi_all
        nout = jnp.maximum(hi_all - wbase, 0) // 16

        def zero_acc(r, c):
            for j in range(_HID // 16):
                wv_acc[pl.ds(r * _HID + j * 16, 16)] = z16f
            z_acc[pl.ds(r * 16, 16)] = z16f
            return c

        lax.fori_loop(0, _BLK + 1, zero_acc, 0)

        def _process(bi, par):
            base = bi * 16

            def edge(e, cc):
                slot = lslot[pl.ds(base + e, 16)][0]
                zb = slot * 16
                wvb = slot * _HID
                zrow = z16f
                for h in range(_H):
                    col = h * 32
                    k0a, k0b = plsc.unpack(
                        plsc.bitcast(krows_d[par, e, pl.ds(col, 16)],
                                     jnp.bfloat16),
                        format=plsc.PackFormat.INTERLEAVED)
                    q0a, q0b = plsc.unpack(
                        plsc.bitcast(qrows_d[par, e, pl.ds(col, 16)],
                                     jnp.bfloat16),
                        format=plsc.PackFormat.INTERLEAVED)
                    k1a, k1b = plsc.unpack(
                        plsc.bitcast(krows_d[par, e, pl.ds(col + 16, 16)],
                                     jnp.bfloat16),
                        format=plsc.PackFormat.INTERLEAVED)
                    q1a, q1b = plsc.unpack(
                        plsc.bitcast(qrows_d[par, e, pl.ds(col + 16, 16)],
                                     jnp.bfloat16),
                        format=plsc.PackFormat.INTERLEAVED)
                    acc = k0a * q0a + k0b * q0b + k1a * q1a + k1b * q1b
                    dot = _allsum(acc)
                    sfull = jnp.exp(jnp.clip(dot * 0.125, -5.0, 5.0))
                    zrow = jnp.where(lane == h, sfull, zrow)
                    for j in range(4):
                        c2 = h * _D + j * 16
                        wv_acc[pl.ds(wvb + c2, 16)] = (
                            wv_acc[pl.ds(wvb + c2, 16)]
                            + vrows_d[par, e, pl.ds(c2, 16)] * sfull)
                z_acc[pl.ds(zb, 16)] = z_acc[pl.ds(zb, 16)] + zrow
                return cc

            lax.fori_loop(0, 16, edge, 0)

        def run_batches(nbt):
            @pl.when(nbt > 0)
            def _():
                _issue(0, jnp.int32(0))

            def batch2(bi, c):
                par = bi % 2
                even = par == 0

                @pl.when(bi + 1 < nbt)
                def _():
                    @pl.when(even)
                    def _():
                        _issue(1, bi + 1)

                    @pl.when(jnp.logical_not(even))
                    def _():
                        _issue(0, bi + 1)

                @pl.when(even)
                def _():
                    _wait(0)

                @pl.when(jnp.logical_not(even))
                def _():
                    _wait(1)

                _process(bi, par)
                return c

            lax.fori_loop(0, nbt, batch2, 0)

        def chunk_body(g, cnt):
            off = g * _ECH
            pltpu.sync_copy(src_hbm.at[pl.ds(off, _ECH)], src_buf)
            pltpu.sync_copy(dst_hbm.at[pl.ds(off, _ECH)], dst_buf)

            def comp(i, cn_v):
                sv = src_buf[pl.ds(i * 16, 16)]
                dv = dst_buf[pl.ds(i * 16, 16)]
                m = (dv >= lo_v) & (dv < hi_v)
                mi = m.astype(jnp.int32)
                excl = plsc.cumsum(mi) - mi
                pos = jnp.where(m, cn_v + excl, dump)
                plsc.store_scatter(lsrc, [pos], sv)
                plsc.store_scatter(ldst, [pos], dv)
                plsc.store_scatter(lslot, [pos], dv - lo_v)
                return cn_v + _allsum(mi)

            cn_v = lax.fori_loop(0, _ECH // 16, comp,
                                 jnp.full((16,), 1, jnp.int32) * cnt)
            cnt = jnp.max(cn_v)
            nbf = cnt // 16
            run_batches(nbf)

            @pl.when(nbf > 0)
            def _():
                mv = nbf * 16
                sv = lsrc[pl.ds(mv, 16)]
                dv = ldst[pl.ds(mv, 16)]
                wv = lslot[pl.ds(mv, 16)]
                lsrc[pl.ds(0, 16)] = sv
                ldst[pl.ds(0, 16)] = dv
                lslot[pl.ds(0, 16)] = wv

            return cnt - nbf * 16

        cnt = lax.fori_loop(0, _E // _ECH, chunk_body, jnp.int32(0))

        lsrc[pl.ds(cnt, 16)] = jnp.zeros((16,), jnp.int32)
        ldst[pl.ds(cnt, 16)] = jnp.zeros((16,), jnp.int32)
        lslot[pl.ds(cnt, 16)] = jnp.full((16,), _TRASH, jnp.int32)
        run_batches((cnt + 15) // 16)

        def out_chunk(ck, c):
            rowbase = ck * 16

            def row(e, cc):
                r = rowbase + e
                zv = z_acc[pl.ds(r * 16, 16)]
                for h in range(_H):
                    zh = zv.at[hvecs[h]].get(mode='promise_in_bounds') + 1e-6
                    for j in range(4):
                        col = h * _D + j * 16
                        vrows_d[0, e, pl.ds(col, 16)] = (
                            wv_acc[pl.ds(r * _HID + col, 16)] / zh)
                return cc

            lax.fori_loop(0, 16, row, 0)
            pltpu.sync_copy(vrows_d.at[0], out_hbm.at[pl.ds(wbase + rowbase, 16)])
            return c

        lax.fori_loop(0, nout, out_chunk, 0)


def kernel(h, edge_index, Wq, bq, Wk, bk, Wv, bv):
    W = jnp.concatenate([Wq, Wk, Wv], axis=1)
    b = jnp.concatenate([bq, bk, bv]).reshape(1, 3 * _HID)
    hp = jnp.pad(h, ((0, _NP - _N), (0, 0)))
    q, k, v = pl.pallas_call(
        _qkv_body,
        grid=(_NP // _BM,),
        in_specs=[
            pl.BlockSpec((_BM, _IN), lambda i: (i, 0)),
            pl.BlockSpec((_IN, 3 * _HID), lambda i: (0, 0)),
            pl.BlockSpec((1, 3 * _HID), lambda i: (0, 0)),
        ],
        out_specs=[pl.BlockSpec((_BM, _HID), lambda i: (i, 0))] * 3,
        out_shape=[
            jax.ShapeDtypeStruct((_NP, _HID), jnp.bfloat16),
            jax.ShapeDtypeStruct((_NP, _HID), jnp.bfloat16),
            jax.ShapeDtypeStruct((_NP, _HID), jnp.float32),
        ],
    )(hp, W, b)
    q32 = lax.bitcast_convert_type(q.reshape(_NP, _HID // 2, 2), jnp.int32)
    k32 = lax.bitcast_convert_type(k.reshape(_NP, _HID // 2, 2), jnp.int32)
    out = _edge_kernel(q32, k32, v, edge_index[0], edge_index[1])
    return out.reshape(_N, _H, _D)


# speedup vs baseline: 1.3444x; 1.3444x over previous
import functools

import jax
import jax.numpy as jnp
from jax import lax
from jax.experimental import pallas as pl
from jax.experimental.pallas import tpu as pltpu
from jax.experimental.pallas import tpu_sc as plsc

_N = 10000
_E = 160000
_H = 8
_D = 64
_HID = _H * _D
_IN = 256

_NP = 10240
_BM = 1024

_NTILE = 32
_BLK = 160
_TRASH = _BLK
_ECH = 4000


def _qkv_body(h_ref, w_ref, b_ref, q_ref, k_ref, v_ref):
    r = jnp.dot(h_ref[...], w_ref[...], preferred_element_type=jnp.float32)
    r = r + b_ref[...]
    q_ref[...] = r[:, :_HID]
    k_ref[...] = r[:, _HID:2 * _HID]
    v_ref[...] = r[:, 2 * _HID:]


_mesh = plsc.VectorSubcoreMesh(core_axis_name="c", subcore_axis_name="s")


@functools.partial(
    pl.kernel,
    out_type=jax.ShapeDtypeStruct((_N, _HID), jnp.float32),
    mesh=_mesh,
    compiler_params=pltpu.CompilerParams(needs_layout_passes=False),
    scratch_types=[
        pltpu.VMEM((_ECH,), jnp.int32),
        pltpu.VMEM((_ECH,), jnp.int32),
        pltpu.VMEM((_ECH + 32,), jnp.int32),
        pltpu.VMEM((_ECH + 32,), jnp.int32),
        pltpu.VMEM((_ECH + 32,), jnp.int32),
        pltpu.VMEM((16,), jnp.int32),
        pltpu.VMEM((16,), jnp.int32),
        pltpu.VMEM((16, _HID), jnp.float32),
        pltpu.VMEM((16, _HID), jnp.float32),
        pltpu.VMEM((16, _HID), jnp.float32),
        pltpu.VMEM(((_BLK + 1) * _HID,), jnp.float32),
        pltpu.VMEM(((_BLK + 1) * 16,), jnp.float32),
        pltpu.SemaphoreType.DMA,
    ],
)
def _edge_kernel(q_hbm, k_hbm, v_hbm, src_hbm, dst_hbm, out_hbm,
                 src_buf, dst_buf, lsrc, ldst, lslot,
                 idx_src, idx_dst,
                 krows, qrows, vrows,
                 wv_acc, z_acc, sem):
    cid = lax.axis_index("c")
    sid = lax.axis_index("s")
    wid = cid * 16 + sid
    lane = jnp.arange(16, dtype=jnp.int32)
    z16f = jnp.zeros((16,), jnp.float32)
    dump = jnp.full((16,), _ECH + 16, jnp.int32)

    for bp in range(2):
        wbase = (bp * _NTILE + wid) * _BLK
        hi_all = jnp.minimum(wbase + _BLK, _N)
        lo_v = jnp.full((16,), 1, jnp.int32) * wbase
        hi_v = jnp.full((16,), 1, jnp.int32) * hi_all
        nout = jnp.maximum(hi_all - wbase, 0) // 16

        def zero_acc(r, c):
            for j in range(_HID // 16):
                wv_acc[pl.ds(r * _HID + j * 16, 16)] = z16f
            z_acc[pl.ds(r * 16, 16)] = z16f
            return c

        lax.fori_loop(0, _BLK + 1, zero_acc, 0)

        def batch(bi, c):
            base = bi * 16
            idx_src[...] = lsrc[pl.ds(base, 16)]
            idx_dst[...] = ldst[pl.ds(base, 16)]
            c1 = pltpu.async_copy(k_hbm.at[idx_src], krows, sem)
            c2 = pltpu.async_copy(q_hbm.at[idx_dst], qrows, sem)
            c3 = pltpu.async_copy(v_hbm.at[idx_src], vrows, sem)
            c1.wait()
            c2.wait()
            c3.wait()

            def edge(e, cc):
                dvec = z16f
                for h in range(_H):
                    col = h * _D
                    acc = krows[e, pl.ds(col, 16)] * qrows[e, pl.ds(col, 16)]
                    for j in range(1, 4):
                        acc = acc + (krows[e, pl.ds(col + j * 16, 16)]
                                     * qrows[e, pl.ds(col + j * 16, 16)])
                    dvec = jnp.where(lane == h, jnp.sum(acc), dvec)
                svec = jnp.exp(jnp.clip(dvec * 0.125, -5.0, 5.0))
                slot = lslot[pl.ds(base + e, 16)][0]
                zb = slot * 16
                z_acc[pl.ds(zb, 16)] = z_acc[pl.ds(zb, 16)] + svec
                wvb = slot * _HID
                for h in range(_H):
                    sh = jnp.sum(jnp.where(lane == h, svec, 0.0))
                    for j in range(4):
                        col = h * _D + j * 16
                        wv_acc[pl.ds(wvb + col, 16)] = (
                            wv_acc[pl.ds(wvb + col, 16)]
                            + vrows[e, pl.ds(col, 16)] * sh)
                return cc

            lax.fori_loop(0, 16, edge, 0)
            return c

        def chunk_body(g, cnt):
            off = g * _ECH
            pltpu.sync_copy(src_hbm.at[pl.ds(off, _ECH)], src_buf)
            pltpu.sync_copy(dst_hbm.at[pl.ds(off, _ECH)], dst_buf)

            def comp(i, cn):
                sv = src_buf[pl.ds(i * 16, 16)]
                dv = dst_buf[pl.ds(i * 16, 16)]
                m = (dv >= lo_v) & (dv < hi_v)
                mi = m.astype(jnp.int32)
                excl = plsc.cumsum(mi) - mi
                pos = jnp.where(m, cn + excl, dump)
                plsc.store_scatter(lsrc, [pos], sv)
                plsc.store_scatter(ldst, [pos], dv)
                plsc.store_scatter(lslot, [pos], dv - lo_v)
                return cn + jnp.sum(mi)

            cnt = lax.fori_loop(0, _ECH // 16, comp, cnt)
            nbf = cnt // 16
            lax.fori_loop(0, nbf, batch, 0)

            @pl.when(nbf > 0)
            def _():
                mv = nbf * 16
                sv = lsrc[pl.ds(mv, 16)]
                dv = ldst[pl.ds(mv, 16)]
                wv = lslot[pl.ds(mv, 16)]
                lsrc[pl.ds(0, 16)] = sv
                ldst[pl.ds(0, 16)] = dv
                lslot[pl.ds(0, 16)] = wv

            return cnt - nbf * 16

        cnt = lax.fori_loop(0, _E // _ECH, chunk_body, jnp.int32(0))

        lsrc[pl.ds(cnt, 16)] = jnp.zeros((16,), jnp.int32)
        ldst[pl.ds(cnt, 16)] = jnp.zeros((16,), jnp.int32)
        lslot[pl.ds(cnt, 16)] = jnp.full((16,), _TRASH, jnp.int32)
        lax.fori_loop(0, (cnt + 15) // 16, batch, 0)

        def out_chunk(ck, c):
            rowbase = ck * 16

            def row(e, cc):
                r = rowbase + e
                zv = z_acc[pl.ds(r * 16, 16)]
                for h in range(_H):
                    zh = jnp.sum(jnp.where(lane == h, zv, 0.0)) + 1e-6
                    for j in range(4):
                        col = h * _D + j * 16
                        vrows[e, pl.ds(col, 16)] = (
                            wv_acc[pl.ds(r * _HID + col, 16)] / zh)
                return cc

            lax.fori_loop(0, 16, row, 0)
            pltpu.sync_copy(vrows, out_hbm.at[pl.ds(wbase + rowbase, 16)])
            return c

        lax.fori_loop(0, nout, out_chunk, 0)


def kernel(h, edge_index, Wq, bq, Wk, bk, Wv, bv):
    W = jnp.concatenate([Wq, Wk, Wv], axis=1)
    b = jnp.concatenate([bq, bk, bv]).reshape(1, 3 * _HID)
    hp = jnp.pad(h, ((0, _NP - _N), (0, 0)))
    q, k, v = pl.pallas_call(
        _qkv_body,
        grid=(_NP // _BM,),
        in_specs=[
            pl.BlockSpec((_BM, _IN), lambda i: (i, 0)),
            pl.BlockSpec((_IN, 3 * _HID), lambda i: (0, 0)),
            pl.BlockSpec((1, 3 * _HID), lambda i: (0, 0)),
        ],
        out_specs=[pl.BlockSpec((_BM, _HID), lambda i: (i, 0))] * 3,
        out_shape=[jax.ShapeDtypeStruct((_NP, _HID), jnp.float32)] * 3,
    )(hp, W, b)
    out = _edge_kernel(q, k, v, edge_index[0], edge_index[1])
    return out.reshape(_N, _H, _D)
